# Initial kernel scaffold; baseline (speedup 1.0000x reference)
import jax
import jax.numpy as jnp
from jax.experimental import pallas as pl


def kernel(x, edge_index, W1, b1, W2, b2):
    raise NotImplementedError("write your pallas kernel here")


# speedup vs baseline: 34.9987x; 34.9987x over previous
import functools

import jax
import jax.numpy as jnp
from jax import lax
from jax.experimental import pallas as pl
from jax.experimental.pallas import tpu as pltpu
from jax.experimental.pallas import tpu_sc as plsc

N = 10000
R = 10240
E = 320000
NT = 32
NB = 80
EP = NT * NB * 128
DUMP = N
DH = 16
DO = 10

_MESH = plsc.VectorSubcoreMesh(
    core_axis_name="c", subcore_axis_name="s", num_cores=2, num_subcores=16
)


@functools.partial(
    pl.kernel,
    out_type=jax.ShapeDtypeStruct((NT, R // 128, 128), jnp.float32),
    mesh=_MESH,
    compiler_params=pltpu.CompilerParams(needs_layout_passes=False),
    scratch_types=[
        pltpu.VMEM((NB, 128), jnp.int32),
        pltpu.VMEM((R // 128, 128), jnp.float32),
    ],
)
def _sc_deg(dst_hbm, degp_hbm, dstv, ldeg):
    c = lax.axis_index("c")
    s = lax.axis_index("s")
    tid = c * 16 + s
    zeros = jnp.zeros((16,), jnp.float32)

    pltpu.sync_copy(dst_hbm.at[tid], dstv)

    def zero_row(i, carry):
        ldeg[i // 8, pl.ds((i % 8) * 16, 16)] = zeros
        return carry

    lax.fori_loop(0, (R // 128) * 8, zero_row, 0)

    def accum(i, carry):
        jj = i // 8
        kk = i % 8
        d = dstv[jj, pl.ds(kk * 16, 16)]
        cnt, last = plsc.scan_count(d)
        plsc.addupdate_scatter(
            ldeg,
            [lax.shift_right_logical(d, 7), lax.bitwise_and(d, 127)],
            cnt.astype(jnp.float32),
            mask=last,
        )
        return carry

    lax.fori_loop(0, NB * 8, accum, 0)
    pltpu.sync_copy(ldeg, degp_hbm.at[tid])


@functools.partial(
    pl.kernel,
    out_type=jax.ShapeDtypeStruct((2, R, DH), jnp.float32),
    mesh=_MESH,
    compiler_params=pltpu.CompilerParams(use_tc_tiling_on_sc=False),
    scratch_types=[
        pltpu.VMEM((NB, 128), jnp.int32),
        pltpu.VMEM((NB, 128), jnp.int32),
        pltpu.VMEM((128, DH), jnp.float32),
        pltpu.VMEM((128, DH), jnp.float32),
        pltpu.SemaphoreType.DMA,
        pltpu.SemaphoreType.DMA,
        pltpu.VMEM_SHARED((R, DH), jnp.float32),
    ],
)
def _sc_msg(hs_hbm, src_hbm, dst_hbm, msgp_hbm, srcv, dstv, rows0, rows1,
            sem0, sem1, acc):
    c = lax.axis_index("c")
    s = lax.axis_index("s")
    tid = c * 16 + s
    zeros = jnp.zeros((16,), jnp.float32)

    pltpu.sync_copy(src_hbm.at[tid], srcv)
    pltpu.sync_copy(dst_hbm.at[tid], dstv)

    def zero_row(i, carry):
        rows0[i, :] = zeros
        return carry

    lax.fori_loop(0, 128, zero_row, 0)
    base = s * (R // 16)

    def zero_sl(q, carry):
        pltpu.sync_copy(rows0, acc.at[pl.ds(base + q * 128, 128)])
        return carry

    lax.fori_loop(0, R // 16 // 128, zero_sl, 0)
    plsc.subcore_barrier()

    def step(jj, carry):
        j0 = 2 * jj
        j1 = j0 + 1
        g0 = pltpu.async_copy(hs_hbm.at[srcv.at[j0]], rows0, sem0)
        g1 = pltpu.async_copy(hs_hbm.at[srcv.at[j1]], rows1, sem1)
        g0.wait()
        pltpu.sync_copy(rows0, acc.at[dstv.at[j0]], add=True)
        g1.wait()
        pltpu.sync_copy(rows1, acc.at[dstv.at[j1]], add=True)
        return carry

    lax.fori_loop(0, NB // 2, step, 0)

    plsc.subcore_barrier()

    @pl.when(s == 0)
    def _():
        pltpu.sync_copy(acc, msgp_hbm.at[c])


def _tc0_body(degp_ref, dinv_ref):
    deg = jnp.sum(degp_ref[...], axis=0) + 1.0
    dinv_ref[...] = lax.rsqrt(jnp.maximum(deg, 1.0))


def _tc1_body(x_ref, w1_ref, dinvc_ref, hs1_ref, dinv_ref):
    dinv16 = jnp.broadcast_to(dinvc_ref[...], (R, DH))
    h = jnp.dot(x_ref[...], w1_ref[...], preferred_element_type=jnp.float32)
    hs1_ref[...] = h * dinv16
    dinv_ref[...] = dinv16


def _tc2_body(msgp_ref, hs1_ref, dinv_ref, b1_ref, w2_ref, hs2_ref):
    msg = msgp_ref[0] + msgp_ref[1]
    pre = dinv_ref[...] * (msg + hs1_ref[...]) + b1_ref[...]
    out1 = jnp.maximum(pre, 0.0)
    h2 = jnp.dot(out1, w2_ref[...], preferred_element_type=jnp.float32)
    hs2_ref[...] = h2 * dinv_ref[...]


def _tc3_body(msgp_ref, hs2_ref, dinv_ref, b2_ref, out_ref):
    msg = msgp_ref[0] + msgp_ref[1]
    z = dinv_ref[...] * (msg + hs2_ref[...]) + b2_ref[...]
    col = lax.broadcasted_iota(jnp.int32, (1, DH), 1)
    z = jnp.where(col < DO, z, -1e30)
    m = jnp.max(z, axis=1, keepdims=True)
    ssum = jnp.sum(jnp.exp(z - m), axis=1, keepdims=True)
    out_ref[...] = z - m - jnp.log(ssum)


def kernel(x, edge_index, W1, b1, W2, b2):
    src = edge_index[0].astype(jnp.int32)
    dst = edge_index[1].astype(jnp.int32)
    pad = EP - E
    srcp = jnp.concatenate([src, jnp.zeros((pad,), jnp.int32)]).reshape(
        NT, NB, 128)
    dstp = jnp.concatenate([dst, jnp.full((pad,), DUMP, jnp.int32)]).reshape(
        NT, NB, 128)
    xp = jnp.pad(x.astype(jnp.float32), ((0, R - N), (0, 0)))
    w2p = jnp.pad(W2, ((0, 0), (0, DH - DO)))
    b1r = b1.reshape(1, DH)
    b2r = jnp.pad(b2, (0, DH - DO)).reshape(1, DH)

    degp = _sc_deg(dstp)

    dinvc = pl.pallas_call(
        _tc0_body,
        out_shape=jax.ShapeDtypeStruct((R // 128, 128), jnp.float32),
    )(degp).reshape(R, 1)

    hs1, dinv = pl.pallas_call(
        _tc1_body,
        out_shape=[
            jax.ShapeDtypeStruct((R, DH), jnp.float32),
            jax.ShapeDtypeStruct((R, DH), jnp.float32),
        ],
    )(xp, W1, dinvc)

    msgp1 = _sc_msg(hs1, srcp, dstp)

    hs2 = pl.pallas_call(
        _tc2_body,
        out_shape=jax.ShapeDtypeStruct((R, DH), jnp.float32),
    )(msgp1, hs1, dinv, b1r, w2p)

    msgp2 = _sc_msg(hs2, srcp, dstp)

    out16 = pl.pallas_call(
        _tc3_body,
        out_shape=jax.ShapeDtypeStruct((R, DH), jnp.float32),
    )(msgp2, hs2, dinv, b2r)

    return out16[:N, :DO]


# speedup vs baseline: 38.1980x; 1.0914x over previous
import functools

import jax
import jax.numpy as jnp
from jax import lax
from jax.experimental import pallas as pl
from jax.experimental.pallas import tpu as pltpu
from jax.experimental.pallas import tpu_sc as plsc

N = 10000
R = 10240
E = 320000
NT = 32
NB = 80
EP = NT * NB * 128
DUMP = N
DH = 16
DO = 10

_MESH = plsc.VectorSubcoreMesh(
    core_axis_name="c", subcore_axis_name="s", num_cores=2, num_subcores=16
)


@functools.partial(
    pl.kernel,
    out_type=jax.ShapeDtypeStruct((NT, R // 128, 128), jnp.float32),
    mesh=_MESH,
    compiler_params=pltpu.CompilerParams(needs_layout_passes=False),
    scratch_types=[
        pltpu.VMEM((NB, 128), jnp.int32),
        pltpu.VMEM((R // 128, 128), jnp.float32),
    ],
)
def _sc_deg(dst_hbm, degp_hbm, dstv, ldeg):
    c = lax.axis_index("c")
    s = lax.axis_index("s")
    tid = c * 16 + s
    zeros = jnp.zeros((16,), jnp.float32)

    pltpu.sync_copy(dst_hbm.at[tid], dstv)

    def zero_row(i, carry):
        ldeg[i // 8, pl.ds((i % 8) * 16, 16)] = zeros
        return carry

    lax.fori_loop(0, (R // 128) * 8, zero_row, 0)

    def accum(i, carry):
        jj = i // 8
        kk = i % 8
        d = dstv[jj, pl.ds(kk * 16, 16)]
        cnt, last = plsc.scan_count(d)
        plsc.addupdate_scatter(
            ldeg,
            [lax.shift_right_logical(d, 7), lax.bitwise_and(d, 127)],
            cnt.astype(jnp.float32),
            mask=last,
        )
        return carry

    lax.fori_loop(0, NB * 8, accum, 0)
    pltpu.sync_copy(ldeg, degp_hbm.at[tid])


@functools.partial(
    pl.kernel,
    out_type=jax.ShapeDtypeStruct((2, R, DH), jnp.float32),
    mesh=_MESH,
    compiler_params=pltpu.CompilerParams(use_tc_tiling_on_sc=False),
    scratch_types=[
        pltpu.VMEM((NB, 128), jnp.int32),
        pltpu.VMEM((NB, 128), jnp.int32),
        pltpu.VMEM((8, 128, DH), jnp.float32),
        pltpu.SemaphoreType.DMA,
        pltpu.SemaphoreType.DMA,
        pltpu.VMEM_SHARED((R, DH), jnp.float32),
    ],
)
def _sc_msg(hs_hbm, src_hbm, dst_hbm, msgp_hbm, srcv, dstv, rows, gsem, ssem,
            acc):
    c = lax.axis_index("c")
    s = lax.axis_index("s")
    tid = c * 16 + s
    zeros = jnp.zeros((16,), jnp.float32)

    pltpu.sync_copy(src_hbm.at[tid], srcv)
    pltpu.sync_copy(dst_hbm.at[tid], dstv)

    def zero_row(i, carry):
        rows[0, i, :] = zeros
        return carry

    lax.fori_loop(0, 128, zero_row, 0)
    base = s * (R // 16)

    def zero_sl(q, carry):
        pltpu.sync_copy(rows.at[0], acc.at[pl.ds(base + q * 128, 128)])
        return carry

    lax.fori_loop(0, R // 16 // 128, zero_sl, 0)
    plsc.subcore_barrier()

    def step(o, carry):
        j = o * 8
        for b in range(8):
            pltpu.async_copy(hs_hbm.at[srcv.at[j + b]], rows.at[b], gsem)
        for b in range(8):
            pltpu.make_async_copy(hs_hbm.at[srcv.at[j + b]], rows.at[b],
                                  gsem).wait()
        for b in range(8):
            pltpu.async_copy(rows.at[b], acc.at[dstv.at[j + b]], ssem,
                             add=True)
        for b in range(8):
            pltpu.make_async_copy(rows.at[b], acc.at[dstv.at[j + b]],
                                  ssem).wait()
        return carry

    lax.fori_loop(0, NB // 8, step, 0)

    plsc.subcore_barrier()

    @pl.when(s == 0)
    def _():
        pltpu.sync_copy(acc, msgp_hbm.at[c])


def _tc0_body(degp_ref, dinv_ref):
    deg = jnp.sum(degp_ref[...], axis=0) + 1.0
    dinv_ref[...] = lax.rsqrt(jnp.maximum(deg, 1.0))


def _tc1_body(x_ref, w1_ref, dinvc_ref, hs1_ref, dinv_ref):
    dinv16 = jnp.broadcast_to(dinvc_ref[...], (R, DH))
    h = jnp.dot(x_ref[...], w1_ref[...], preferred_element_type=jnp.float32)
    hs1_ref[...] = h * dinv16
    dinv_ref[...] = dinv16


def _tc2_body(msgp_ref, hs1_ref, dinv_ref, b1_ref, w2_ref, hs2_ref):
    msg = msgp_ref[0] + msgp_ref[1]
    pre = dinv_ref[...] * (msg + hs1_ref[...]) + b1_ref[...]
    out1 = jnp.maximum(pre, 0.0)
    h2 = jnp.dot(out1, w2_ref[...], preferred_element_type=jnp.float32)
    hs2_ref[...] = h2 * dinv_ref[...]


def _tc3_body(msgp_ref, hs2_ref, dinv_ref, b2_ref, out_ref):
    msg = msgp_ref[0] + msgp_ref[1]
    z = dinv_ref[...] * (msg + hs2_ref[...]) + b2_ref[...]
    col = lax.broadcasted_iota(jnp.int32, (1, DH), 1)
    z = jnp.where(col < DO, z, -1e30)
    m = jnp.max(z, axis=1, keepdims=True)
    ssum = jnp.sum(jnp.exp(z - m), axis=1, keepdims=True)
    out_ref[...] = z - m - jnp.log(ssum)


def kernel(x, edge_index, W1, b1, W2, b2):
    src = edge_index[0].astype(jnp.int32)
    dst = edge_index[1].astype(jnp.int32)
    pad = EP - E
    srcp = jnp.concatenate([src, jnp.zeros((pad,), jnp.int32)]).reshape(
        NT, NB, 128)
    dstp = jnp.concatenate([dst, jnp.full((pad,), DUMP, jnp.int32)]).reshape(
        NT, NB, 128)
    xp = jnp.pad(x.astype(jnp.float32), ((0, R - N), (0, 0)))
    w2p = jnp.pad(W2, ((0, 0), (0, DH - DO)))
    b1r = b1.reshape(1, DH)
    b2r = jnp.pad(b2, (0, DH - DO)).reshape(1, DH)

    degp = _sc_deg(dstp)

    dinvc = pl.pallas_call(
        _tc0_body,
        out_shape=jax.ShapeDtypeStruct((R // 128, 128), jnp.float32),
    )(degp).reshape(R, 1)

    hs1, dinv = pl.pallas_call(
        _tc1_body,
        out_shape=[
            jax.ShapeDtypeStruct((R, DH), jnp.float32),
            jax.ShapeDtypeStruct((R, DH), jnp.float32),
        ],
    )(xp, W1, dinvc)

    msgp1 = _sc_msg(hs1, srcp, dstp)

    hs2 = pl.pallas_call(
        _tc2_body,
        out_shape=jax.ShapeDtypeStruct((R, DH), jnp.float32),
    )(msgp1, hs1, dinv, b1r, w2p)

    msgp2 = _sc_msg(hs2, srcp, dstp)

    out16 = pl.pallas_call(
        _tc3_body,
        out_shape=jax.ShapeDtypeStruct((R, DH), jnp.float32),
    )(msgp2, hs2, dinv, b2r)

    return out16[:N, :DO]


# speedup vs baseline: 53.5644x; 1.4023x over previous
import functools

import jax
import jax.numpy as jnp
from jax import lax
from jax.experimental import pallas as pl
from jax.experimental.pallas import tpu as pltpu
from jax.experimental.pallas import tpu_sc as plsc

N = 10000
R = 10240
E = 320000
NT = 32
NB = 80
EP = NT * NB * 128
DUMP = N
DH = 16
DO = 10

_MESH = plsc.VectorSubcoreMesh(
    core_axis_name="c", subcore_axis_name="s", num_cores=2, num_subcores=16
)


@functools.partial(
    pl.kernel,
    out_type=jax.ShapeDtypeStruct((NT, R // 128, 128), jnp.float32),
    mesh=_MESH,
    compiler_params=pltpu.CompilerParams(needs_layout_passes=False),
    scratch_types=[
        pltpu.VMEM((NB, 128), jnp.int32),
        pltpu.VMEM((R // 128, 128), jnp.float32),
    ],
)
def _sc_deg(dst_hbm, degp_hbm, dstv, ldeg):
    c = lax.axis_index("c")
    s = lax.axis_index("s")
    tid = c * 16 + s
    zeros = jnp.zeros((16,), jnp.float32)

    pltpu.sync_copy(dst_hbm.at[tid], dstv)

    def zero_row(i, carry):
        ldeg[i // 8, pl.ds((i % 8) * 16, 16)] = zeros
        return carry

    lax.fori_loop(0, (R // 128) * 8, zero_row, 0)

    def accum(i, carry):
        jj = i // 8
        kk = i % 8
        d = dstv[jj, pl.ds(kk * 16, 16)]
        cnt, last = plsc.scan_count(d)
        plsc.addupdate_scatter(
            ldeg,
            [lax.shift_right_logical(d, 7), lax.bitwise_and(d, 127)],
            cnt.astype(jnp.float32),
            mask=last,
        )
        return carry

    lax.fori_loop(0, NB * 8, accum, 0)
    pltpu.sync_copy(ldeg, degp_hbm.at[tid])


@functools.partial(
    pl.kernel,
    out_type=jax.ShapeDtypeStruct((2, R, DH), jnp.float32),
    mesh=_MESH,
    compiler_params=pltpu.CompilerParams(use_tc_tiling_on_sc=False),
    scratch_types=[
        pltpu.VMEM((NB, 128), jnp.int32),
        pltpu.VMEM((NB, 128), jnp.int32),
        pltpu.VMEM((8, 128, DH), jnp.float32),
        pltpu.SemaphoreType.DMA,
        pltpu.SemaphoreType.DMA,
        pltpu.VMEM_SHARED((R, DH), jnp.float32),
        pltpu.VMEM_SHARED((R, DH), jnp.float32),
    ],
)
def _sc_msg(hs_hbm, src_hbm, dst_hbm, msgp_hbm, srcv, dstv, rows, gsem, ssem,
            acc, hs_sh):
    c = lax.axis_index("c")
    s = lax.axis_index("s")
    tid = c * 16 + s
    zeros = jnp.zeros((16,), jnp.float32)

    pltpu.sync_copy(src_hbm.at[tid], srcv)
    pltpu.sync_copy(dst_hbm.at[tid], dstv)
    base = s * (R // 16)
    pltpu.sync_copy(hs_hbm.at[pl.ds(base, R // 16)],
                    hs_sh.at[pl.ds(base, R // 16)])

    def zero_row(i, carry):
        rows[0, i, :] = zeros
        return carry

    lax.fori_loop(0, 128, zero_row, 0)

    def zero_sl(q, carry):
        pltpu.sync_copy(rows.at[0], acc.at[pl.ds(base + q * 128, 128)])
        return carry

    lax.fori_loop(0, R // 16 // 128, zero_sl, 0)
    plsc.subcore_barrier()

    def step(o, carry):
        j = o * 8
        for b in range(8):
            pltpu.async_copy(hs_sh.at[srcv.at[j + b]], rows.at[b], gsem)
        for b in range(8):
            pltpu.make_async_copy(hs_sh.at[srcv.at[j + b]], rows.at[b],
                                  gsem).wait()
        for b in range(8):
            pltpu.async_copy(rows.at[b], acc.at[dstv.at[j + b]], ssem,
                             add=True)
        for b in range(8):
            pltpu.make_async_copy(rows.at[b], acc.at[dstv.at[j + b]],
                                  ssem).wait()
        return carry

    lax.fori_loop(0, NB // 8, step, 0)

    plsc.subcore_barrier()

    @pl.when(s == 0)
    def _():
        pltpu.sync_copy(acc, msgp_hbm.at[c])


def _tc0_body(degp_ref, dinv_ref):
    deg = jnp.sum(degp_ref[...], axis=0) + 1.0
    dinv_ref[...] = lax.rsqrt(jnp.maximum(deg, 1.0))


def _tc1_body(x_ref, w1_ref, dinvc_ref, hs1_ref, dinv_ref):
    dinv16 = jnp.broadcast_to(dinvc_ref[...], (R, DH))
    h = jnp.dot(x_ref[...], w1_ref[...], preferred_element_type=jnp.float32)
    hs1_ref[...] = h * dinv16
    dinv_ref[...] = dinv16


def _tc2_body(msgp_ref, hs1_ref, dinv_ref, b1_ref, w2_ref, hs2_ref):
    msg = msgp_ref[0] + msgp_ref[1]
    pre = dinv_ref[...] * (msg + hs1_ref[...]) + b1_ref[...]
    out1 = jnp.maximum(pre, 0.0)
    h2 = jnp.dot(out1, w2_ref[...], preferred_element_type=jnp.float32)
    hs2_ref[...] = h2 * dinv_ref[...]


def _tc3_body(msgp_ref, hs2_ref, dinv_ref, b2_ref, out_ref):
    msg = msgp_ref[0] + msgp_ref[1]
    z = dinv_ref[...] * (msg + hs2_ref[...]) + b2_ref[...]
    col = lax.broadcasted_iota(jnp.int32, (1, DH), 1)
    z = jnp.where(col < DO, z, -1e30)
    m = jnp.max(z, axis=1, keepdims=True)
    ssum = jnp.sum(jnp.exp(z - m), axis=1, keepdims=True)
    out_ref[...] = z - m - jnp.log(ssum)


def kernel(x, edge_index, W1, b1, W2, b2):
    src = edge_index[0].astype(jnp.int32)
    dst = edge_index[1].astype(jnp.int32)
    pad = EP - E
    srcp = jnp.concatenate([src, jnp.zeros((pad,), jnp.int32)]).reshape(
        NT, NB, 128)
    dstp = jnp.concatenate([dst, jnp.full((pad,), DUMP, jnp.int32)]).reshape(
        NT, NB, 128)
    xp = jnp.pad(x.astype(jnp.float32), ((0, R - N), (0, 0)))
    w2p = jnp.pad(W2, ((0, 0), (0, DH - DO)))
    b1r = b1.reshape(1, DH)
    b2r = jnp.pad(b2, (0, DH - DO)).reshape(1, DH)

    degp = _sc_deg(dstp)

    dinvc = pl.pallas_call(
        _tc0_body,
        out_shape=jax.ShapeDtypeStruct((R // 128, 128), jnp.float32),
    )(degp).reshape(R, 1)

    hs1, dinv = pl.pallas_call(
        _tc1_body,
        out_shape=[
            jax.ShapeDtypeStruct((R, DH), jnp.float32),
            jax.ShapeDtypeStruct((R, DH), jnp.float32),
        ],
    )(xp, W1, dinvc)

    msgp1 = _sc_msg(hs1, srcp, dstp)

    hs2 = pl.pallas_call(
        _tc2_body,
        out_shape=jax.ShapeDtypeStruct((R, DH), jnp.float32),
    )(msgp1, hs1, dinv, b1r, w2p)

    msgp2 = _sc_msg(hs2, srcp, dstp)

    out16 = pl.pallas_call(
        _tc3_body,
        out_shape=jax.ShapeDtypeStruct((R, DH), jnp.float32),
    )(msgp2, hs2, dinv, b2r)

    return out16[:N, :DO]


# speedup vs baseline: 54.3601x; 1.0149x over previous
import functools

import jax
import jax.numpy as jnp
from jax import lax
from jax.experimental import pallas as pl
from jax.experimental.pallas import tpu as pltpu
from jax.experimental.pallas import tpu_sc as plsc

N = 10000
R = 10240
E = 320000
NT = 32
NB = 80
EP = NT * NB * 128
DUMP = N
DH = 16
DO = 10

_MESH = plsc.VectorSubcoreMesh(
    core_axis_name="c", subcore_axis_name="s", num_cores=2, num_subcores=16
)


@functools.partial(
    pl.kernel,
    out_type=jax.ShapeDtypeStruct((NT, R // 128, 128), jnp.float32),
    mesh=_MESH,
    compiler_params=pltpu.CompilerParams(needs_layout_passes=False),
    scratch_types=[
        pltpu.VMEM((NB, 128), jnp.int32),
        pltpu.VMEM((R // 128, 128), jnp.float32),
    ],
)
def _sc_deg(dst_hbm, degp_hbm, dstv, ldeg):
    c = lax.axis_index("c")
    s = lax.axis_index("s")
    tid = c * 16 + s
    zeros = jnp.zeros((16,), jnp.float32)

    pltpu.sync_copy(dst_hbm.at[tid], dstv)

    def zero_row(i, carry):
        for kk in range(8):
            ldeg[i, pl.ds(kk * 16, 16)] = zeros
        return carry

    lax.fori_loop(0, R // 128, zero_row, 0)

    def accum(jj, carry):
        for kk in range(8):
            d = dstv[jj, pl.ds(kk * 16, 16)]
            cnt, last = plsc.scan_count(d)
            plsc.addupdate_scatter(
                ldeg,
                [lax.shift_right_logical(d, 7), lax.bitwise_and(d, 127)],
                cnt.astype(jnp.float32),
                mask=last,
            )
        return carry

    lax.fori_loop(0, NB, accum, 0)
    pltpu.sync_copy(ldeg, degp_hbm.at[tid])


@functools.partial(
    pl.kernel,
    out_type=jax.ShapeDtypeStruct((2, R, DH), jnp.float32),
    mesh=_MESH,
    compiler_params=pltpu.CompilerParams(use_tc_tiling_on_sc=False),
    scratch_types=[
        pltpu.VMEM((NB, 128), jnp.int32),
        pltpu.VMEM((NB, 128), jnp.int32),
        pltpu.VMEM((8, 128, DH), jnp.float32),
        pltpu.SemaphoreType.DMA,
        pltpu.SemaphoreType.DMA,
        pltpu.VMEM_SHARED((R, DH), jnp.float32),
        pltpu.VMEM_SHARED((R, DH), jnp.float32),
    ],
)
def _sc_msg(hs_hbm, src_hbm, dst_hbm, msgp_hbm, srcv, dstv, rows, gsem, ssem,
            acc, hs_sh):
    c = lax.axis_index("c")
    s = lax.axis_index("s")
    tid = c * 16 + s
    zeros = jnp.zeros((16,), jnp.float32)

    pltpu.sync_copy(src_hbm.at[tid], srcv)
    pltpu.sync_copy(dst_hbm.at[tid], dstv)
    base = s * (R // 16)
    pltpu.sync_copy(hs_hbm.at[pl.ds(base, R // 16)],
                    hs_sh.at[pl.ds(base, R // 16)])

    def zero_row(i, carry):
        rows[0, i, :] = zeros
        return carry

    lax.fori_loop(0, 128, zero_row, 0)

    def zero_sl(q, carry):
        pltpu.sync_copy(rows.at[0], acc.at[pl.ds(base + q * 128, 128)])
        return carry

    lax.fori_loop(0, R // 16 // 128, zero_sl, 0)
    plsc.subcore_barrier()

    def step(o, carry):
        j = o * 8
        for b in range(8):
            pltpu.async_copy(hs_sh.at[srcv.at[j + b]], rows.at[b], gsem)
        for b in range(8):
            pltpu.make_async_copy(hs_sh.at[srcv.at[j + b]], rows.at[b],
                                  gsem).wait()
        for b in range(8):
            pltpu.async_copy(rows.at[b], acc.at[dstv.at[j + b]], ssem,
                             add=True)
        for b in range(8):
            pltpu.make_async_copy(rows.at[b], acc.at[dstv.at[j + b]],
                                  ssem).wait()
        return carry

    lax.fori_loop(0, NB // 8, step, 0)

    plsc.subcore_barrier()

    @pl.when(s == 0)
    def _():
        pltpu.sync_copy(acc, msgp_hbm.at[c])


def _tc0_body(degp_ref, dinv_ref):
    deg = jnp.sum(degp_ref[...], axis=0) + 1.0
    dinv_ref[...] = lax.rsqrt(jnp.maximum(deg, 1.0))


def _tc1_body(x_ref, w1_ref, dinvc_ref, hs1_ref, dinv_ref):
    dinv16 = jnp.broadcast_to(dinvc_ref[...], (R, DH))
    h = jnp.dot(x_ref[...], w1_ref[...], preferred_element_type=jnp.float32)
    hs1_ref[...] = h * dinv16
    dinv_ref[...] = dinv16


def _tc2_body(msgp_ref, hs1_ref, dinv_ref, b1_ref, w2_ref, hs2_ref):
    msg = msgp_ref[0] + msgp_ref[1]
    pre = dinv_ref[...] * (msg + hs1_ref[...]) + b1_ref[...]
    out1 = jnp.maximum(pre, 0.0)
    h2 = jnp.dot(out1, w2_ref[...], preferred_element_type=jnp.float32)
    hs2_ref[...] = h2 * dinv_ref[...]


def _tc3_body(msgp_ref, hs2_ref, dinv_ref, b2_ref, out_ref):
    msg = msgp_ref[0] + msgp_ref[1]
    z = dinv_ref[...] * (msg + hs2_ref[...]) + b2_ref[...]
    col = lax.broadcasted_iota(jnp.int32, (1, DH), 1)
    z = jnp.where(col < DO, z, -1e30)
    m = jnp.max(z, axis=1, keepdims=True)
    ssum = jnp.sum(jnp.exp(z - m), axis=1, keepdims=True)
    out_ref[...] = z - m - jnp.log(ssum)


def kernel(x, edge_index, W1, b1, W2, b2):
    src = edge_index[0].astype(jnp.int32)
    dst = edge_index[1].astype(jnp.int32)
    pad = EP - E
    srcp = jnp.concatenate([src, jnp.zeros((pad,), jnp.int32)]).reshape(
        NT, NB, 128)
    dstp = jnp.concatenate([dst, jnp.full((pad,), DUMP, jnp.int32)]).reshape(
        NT, NB, 128)
    xp = jnp.pad(x.astype(jnp.float32), ((0, R - N), (0, 0)))
    w2p = jnp.pad(W2, ((0, 0), (0, DH - DO)))
    b1r = b1.reshape(1, DH)
    b2r = jnp.pad(b2, (0, DH - DO)).reshape(1, DH)

    degp = _sc_deg(dstp)

    dinvc = pl.pallas_call(
        _tc0_body,
        out_shape=jax.ShapeDtypeStruct((R // 128, 128), jnp.float32),
    )(degp).reshape(R, 1)

    hs1, dinv = pl.pallas_call(
        _tc1_body,
        out_shape=[
            jax.ShapeDtypeStruct((R, DH), jnp.float32),
            jax.ShapeDtypeStruct((R, DH), jnp.float32),
        ],
    )(xp, W1, dinvc)

    msgp1 = _sc_msg(hs1, srcp, dstp)

    hs2 = pl.pallas_call(
        _tc2_body,
        out_shape=jax.ShapeDtypeStruct((R, DH), jnp.float32),
    )(msgp1, hs1, dinv, b1r, w2p)

    msgp2 = _sc_msg(hs2, srcp, dstp)

    out16 = pl.pallas_call(
        _tc3_body,
        out_shape=jax.ShapeDtypeStruct((R, DH), jnp.float32),
    )(msgp2, hs2, dinv, b2r)

    return out16[:N, :DO]


# speedup vs baseline: 57.4900x; 1.0576x over previous
import functools

import jax
import jax.numpy as jnp
from jax import lax
from jax.experimental import pallas as pl
from jax.experimental.pallas import tpu as pltpu
from jax.experimental.pallas import tpu_sc as plsc

N = 10000
R = 10240
E = 320000
NT = 32
NB = 80
EP = NT * NB * 128
DUMP = N
DH = 16
DO = 10

_MESH = plsc.VectorSubcoreMesh(
    core_axis_name="c", subcore_axis_name="s", num_cores=2, num_subcores=16
)


@functools.partial(
    pl.kernel,
    out_type=jax.ShapeDtypeStruct((NT, R // 128, 128), jnp.float32),
    mesh=_MESH,
    compiler_params=pltpu.CompilerParams(needs_layout_passes=False),
    scratch_types=[
        pltpu.VMEM((NB, 128), jnp.int32),
        pltpu.VMEM((R // 128, 128), jnp.float32),
    ],
)
def _sc_deg(dst_hbm, degp_hbm, dstv, ldeg):
    c = lax.axis_index("c")
    s = lax.axis_index("s")
    tid = c * 16 + s
    zeros = jnp.zeros((16,), jnp.float32)

    pltpu.sync_copy(dst_hbm.at[tid], dstv)

    def zero_row(i, carry):
        for kk in range(8):
            ldeg[i, pl.ds(kk * 16, 16)] = zeros
        return carry

    lax.fori_loop(0, R // 128, zero_row, 0)

    def accum(jj, carry):
        for kk in range(8):
            d = dstv[jj, pl.ds(kk * 16, 16)]
            cnt, last = plsc.scan_count(d)
            plsc.addupdate_scatter(
                ldeg,
                [lax.shift_right_logical(d, 7), lax.bitwise_and(d, 127)],
                cnt.astype(jnp.float32),
                mask=last,
            )
        return carry

    lax.fori_loop(0, NB, accum, 0)
    pltpu.sync_copy(ldeg, degp_hbm.at[tid])


@functools.partial(
    pl.kernel,
    out_type=jax.ShapeDtypeStruct((2, R, DH), jnp.float32),
    mesh=_MESH,
    compiler_params=pltpu.CompilerParams(use_tc_tiling_on_sc=False),
    scratch_types=[
        pltpu.VMEM((NB, 128), jnp.int32),
        pltpu.VMEM((NB, 128), jnp.int32),
        pltpu.VMEM((16, 128, DH), jnp.float32),
        pltpu.SemaphoreType.DMA,
        pltpu.SemaphoreType.DMA,
        pltpu.VMEM_SHARED((R, DH), jnp.float32),
        pltpu.VMEM_SHARED((R, DH), jnp.float32),
    ],
)
def _sc_msg(hs_hbm, src_hbm, dst_hbm, msgp_hbm, srcv, dstv, rows, gsem, ssem,
            acc, hs_sh):
    c = lax.axis_index("c")
    s = lax.axis_index("s")
    tid = c * 16 + s
    zeros = jnp.zeros((16,), jnp.float32)

    pltpu.sync_copy(src_hbm.at[tid], srcv)
    pltpu.sync_copy(dst_hbm.at[tid], dstv)
    base = s * (R // 16)
    pltpu.sync_copy(hs_hbm.at[pl.ds(base, R // 16)],
                    hs_sh.at[pl.ds(base, R // 16)])

    def zero_row(i, carry):
        rows[0, i, :] = zeros
        return carry

    lax.fori_loop(0, 128, zero_row, 0)

    def zero_sl(q, carry):
        pltpu.sync_copy(rows.at[0], acc.at[pl.ds(base + q * 128, 128)])
        return carry

    lax.fori_loop(0, R // 16 // 128, zero_sl, 0)
    plsc.subcore_barrier()

    def gather8(j, base, sem):
        for b in range(8):
            pltpu.async_copy(hs_sh.at[srcv.at[j + b]], rows.at[base + b], sem)

    def drain8(j, base, sem):
        for b in range(8):
            pltpu.make_async_copy(hs_sh.at[srcv.at[j + b]], rows.at[base + b],
                                  sem).wait()

    def scatter8(j, base, sem):
        for b in range(8):
            pltpu.async_copy(rows.at[base + b], acc.at[dstv.at[j + b]], sem,
                             add=True)

    def sdrain8(j, base, sem):
        for b in range(8):
            pltpu.make_async_copy(rows.at[base + b], acc.at[dstv.at[j + b]],
                                  sem).wait()

    gather8(0, 0, gsem)
    nsteps = NB // 16

    def step(o, carry):
        jA = o * 16
        jB = jA + 8
        drain8(jA, 0, gsem)
        scatter8(jA, 0, ssem)
        gather8(jB, 8, gsem)
        sdrain8(jA, 0, ssem)
        drain8(jB, 8, gsem)
        scatter8(jB, 8, ssem)

        @pl.when(o < nsteps - 1)
        def _():
            gather8(jA + 16, 0, gsem)

        sdrain8(jB, 8, ssem)
        return carry

    lax.fori_loop(0, nsteps, step, 0)

    plsc.subcore_barrier()

    @pl.when(s == 0)
    def _():
        pltpu.sync_copy(acc, msgp_hbm.at[c])


def _tc0_body(degp_ref, dinv_ref):
    deg = jnp.sum(degp_ref[...], axis=0) + 1.0
    dinv_ref[...] = lax.rsqrt(jnp.maximum(deg, 1.0))


def _tc1_body(x_ref, w1_ref, dinvc_ref, hs1_ref, dinv_ref):
    dinv16 = jnp.broadcast_to(dinvc_ref[...], (R, DH))
    h = jnp.dot(x_ref[...], w1_ref[...], preferred_element_type=jnp.float32)
    hs1_ref[...] = h * dinv16
    dinv_ref[...] = dinv16


def _tc2_body(msgp_ref, hs1_ref, dinv_ref, b1_ref, w2_ref, hs2_ref):
    msg = msgp_ref[0] + msgp_ref[1]
    pre = dinv_ref[...] * (msg + hs1_ref[...]) + b1_ref[...]
    out1 = jnp.maximum(pre, 0.0)
    h2 = jnp.dot(out1, w2_ref[...], preferred_element_type=jnp.float32)
    hs2_ref[...] = h2 * dinv_ref[...]


def _tc3_body(msgp_ref, hs2_ref, dinv_ref, b2_ref, out_ref):
    msg = msgp_ref[0] + msgp_ref[1]
    z = dinv_ref[...] * (msg + hs2_ref[...]) + b2_ref[...]
    col = lax.broadcasted_iota(jnp.int32, (1, DH), 1)
    z = jnp.where(col < DO, z, -1e30)
    m = jnp.max(z, axis=1, keepdims=True)
    ssum = jnp.sum(jnp.exp(z - m), axis=1, keepdims=True)
    out_ref[...] = z - m - jnp.log(ssum)


def kernel(x, edge_index, W1, b1, W2, b2):
    src = edge_index[0].astype(jnp.int32)
    dst = edge_index[1].astype(jnp.int32)
    pad = EP - E
    srcp = jnp.concatenate([src, jnp.zeros((pad,), jnp.int32)]).reshape(
        NT, NB, 128)
    dstp = jnp.concatenate([dst, jnp.full((pad,), DUMP, jnp.int32)]).reshape(
        NT, NB, 128)
    xp = jnp.pad(x.astype(jnp.float32), ((0, R - N), (0, 0)))
    w2p = jnp.pad(W2, ((0, 0), (0, DH - DO)))
    b1r = b1.reshape(1, DH)
    b2r = jnp.pad(b2, (0, DH - DO)).reshape(1, DH)

    degp = _sc_deg(dstp)

    dinvc = pl.pallas_call(
        _tc0_body,
        out_shape=jax.ShapeDtypeStruct((R // 128, 128), jnp.float32),
    )(degp).reshape(R, 1)

    hs1, dinv = pl.pallas_call(
        _tc1_body,
        out_shape=[
            jax.ShapeDtypeStruct((R, DH), jnp.float32),
            jax.ShapeDtypeStruct((R, DH), jnp.float32),
        ],
    )(xp, W1, dinvc)

    msgp1 = _sc_msg(hs1, srcp, dstp)

    hs2 = pl.pallas_call(
        _tc2_body,
        out_shape=jax.ShapeDtypeStruct((R, DH), jnp.float32),
    )(msgp1, hs1, dinv, b1r, w2p)

    msgp2 = _sc_msg(hs2, srcp, dstp)

    out16 = pl.pallas_call(
        _tc3_body,
        out_shape=jax.ShapeDtypeStruct((R, DH), jnp.float32),
    )(msgp2, hs2, dinv, b2r)

    return out16[:N, :DO]


# speedup vs baseline: 57.7304x; 1.0042x over previous
import functools

import jax
import jax.numpy as jnp
from jax import lax
from jax.experimental import pallas as pl
from jax.experimental.pallas import tpu as pltpu
from jax.experimental.pallas import tpu_sc as plsc

N = 10000
R = 10240
E = 320000
NT = 32
NB = 80
EP = NT * NB * 128
DUMP = N
DH = 16
DO = 10

_MESH = plsc.VectorSubcoreMesh(
    core_axis_name="c", subcore_axis_name="s", num_cores=2, num_subcores=16
)


@functools.partial(
    pl.kernel,
    out_type=jax.ShapeDtypeStruct((NT, R // 128, 128), jnp.float32),
    mesh=_MESH,
    compiler_params=pltpu.CompilerParams(needs_layout_passes=False),
    scratch_types=[
        pltpu.VMEM((NB, 128), jnp.int32),
        pltpu.VMEM((R // 128, 128), jnp.float32),
    ],
)
def _sc_deg(dst_hbm, degp_hbm, dstv, ldeg):
    c = lax.axis_index("c")
    s = lax.axis_index("s")
    tid = c * 16 + s
    zeros = jnp.zeros((16,), jnp.float32)

    pltpu.sync_copy(dst_hbm.at[tid], dstv)

    def zero_row(i, carry):
        for kk in range(8):
            ldeg[i, pl.ds(kk * 16, 16)] = zeros
        return carry

    lax.fori_loop(0, R // 128, zero_row, 0)

    def accum(jj, carry):
        for kk in range(8):
            d = dstv[jj, pl.ds(kk * 16, 16)]
            cnt, last = plsc.scan_count(d)
            plsc.addupdate_scatter(
                ldeg,
                [lax.shift_right_logical(d, 7), lax.bitwise_and(d, 127)],
                cnt.astype(jnp.float32),
                mask=last,
            )
        return carry

    lax.fori_loop(0, NB, accum, 0)
    pltpu.sync_copy(ldeg, degp_hbm.at[tid])


@functools.partial(
    pl.kernel,
    out_type=jax.ShapeDtypeStruct((2, R, DH), jnp.float32),
    mesh=_MESH,
    compiler_params=pltpu.CompilerParams(use_tc_tiling_on_sc=False),
    scratch_types=[
        pltpu.VMEM((NB, 128), jnp.int32),
        pltpu.VMEM((NB, 128), jnp.int32),
        pltpu.VMEM((16, 128, DH), jnp.float32),
        pltpu.SemaphoreType.DMA,
        pltpu.SemaphoreType.DMA,
        pltpu.VMEM_SHARED((R, DH), jnp.float32),
        pltpu.VMEM_SHARED((R, DH), jnp.float32),
    ],
)
def _sc_msg(hs_hbm, src_hbm, dst_hbm, msgp_hbm, srcv, dstv, rows, gsem, ssem,
            acc, hs_sh):
    c = lax.axis_index("c")
    s = lax.axis_index("s")
    tid = c * 16 + s
    zeros = jnp.zeros((16,), jnp.float32)

    pltpu.sync_copy(src_hbm.at[tid], srcv)
    pltpu.sync_copy(dst_hbm.at[tid], dstv)
    base = s * (R // 16)
    pltpu.sync_copy(hs_hbm.at[pl.ds(base, R // 16)],
                    hs_sh.at[pl.ds(base, R // 16)])

    def zero_row(i, carry):
        rows[0, i, :] = zeros
        return carry

    lax.fori_loop(0, 128, zero_row, 0)

    def zero_sl(q, carry):
        pltpu.sync_copy(rows.at[0], acc.at[pl.ds(base + q * 128, 128)])
        return carry

    lax.fori_loop(0, R // 16 // 128, zero_sl, 0)
    plsc.subcore_barrier()

    def gather8(j, base, sem):
        for b in range(8):
            pltpu.async_copy(hs_sh.at[srcv.at[j + b]], rows.at[base + b], sem)

    def drain8(j, base, sem):
        for b in range(8):
            pltpu.make_async_copy(hs_sh.at[srcv.at[j + b]], rows.at[base + b],
                                  sem).wait()

    def scatter8(j, base, sem):
        for b in range(8):
            pltpu.async_copy(rows.at[base + b], acc.at[dstv.at[j + b]], sem,
                             add=True)

    def sdrain8(j, base, sem):
        for b in range(8):
            pltpu.make_async_copy(rows.at[base + b], acc.at[dstv.at[j + b]],
                                  sem).wait()

    gather8(0, 0, gsem)
    nsteps = NB // 16

    def step(o, carry):
        jA = o * 16
        jB = jA + 8
        drain8(jA, 0, gsem)
        scatter8(jA, 0, ssem)
        gather8(jB, 8, gsem)
        sdrain8(jA, 0, ssem)
        drain8(jB, 8, gsem)
        scatter8(jB, 8, ssem)

        @pl.when(o < nsteps - 1)
        def _():
            gather8(jA + 16, 0, gsem)

        sdrain8(jB, 8, ssem)
        return carry

    lax.fori_loop(0, nsteps, step, 0)

    plsc.subcore_barrier()

    @pl.when(s == 0)
    def _():
        pltpu.sync_copy(acc, msgp_hbm.at[c])


def _tc0_body(degp_ref, dinv_ref):
    deg = jnp.sum(degp_ref[...], axis=0) + 1.0
    dinv_ref[...] = lax.rsqrt(jnp.maximum(deg, 1.0))


def _tc_mm_body(x_ref, w1_ref, h_ref):
    h_ref[...] = jnp.dot(x_ref[...], w1_ref[...],
                         preferred_element_type=jnp.float32)


def _tc1_body(h_ref, dinvc_ref, hs1_ref, dinv_ref):
    dinv16 = jnp.broadcast_to(dinvc_ref[...], (R, DH))
    hs1_ref[...] = h_ref[...] * dinv16
    dinv_ref[...] = dinv16


def _tc2_body(msgp_ref, hs1_ref, dinv_ref, b1_ref, w2_ref, hs2_ref):
    msg = msgp_ref[0] + msgp_ref[1]
    pre = dinv_ref[...] * (msg + hs1_ref[...]) + b1_ref[...]
    out1 = jnp.maximum(pre, 0.0)
    h2 = jnp.dot(out1, w2_ref[...], preferred_element_type=jnp.float32)
    hs2_ref[...] = h2 * dinv_ref[...]


def _tc3_body(msgp_ref, hs2_ref, dinv_ref, b2_ref, out_ref):
    msg = msgp_ref[0] + msgp_ref[1]
    z = dinv_ref[...] * (msg + hs2_ref[...]) + b2_ref[...]
    col = lax.broadcasted_iota(jnp.int32, (1, DH), 1)
    z = jnp.where(col < DO, z, -1e30)
    m = jnp.max(z, axis=1, keepdims=True)
    ssum = jnp.sum(jnp.exp(z - m), axis=1, keepdims=True)
    out_ref[...] = z - m - jnp.log(ssum)


def kernel(x, edge_index, W1, b1, W2, b2):
    src = edge_index[0].astype(jnp.int32)
    dst = edge_index[1].astype(jnp.int32)
    pad = EP - E
    srcp = jnp.concatenate([src, jnp.zeros((pad,), jnp.int32)]).reshape(
        NT, NB, 128)
    dstp = jnp.concatenate([dst, jnp.full((pad,), DUMP, jnp.int32)]).reshape(
        NT, NB, 128)
    xp = jnp.pad(x.astype(jnp.float32), ((0, R - N), (0, 0)))
    w2p = jnp.pad(W2, ((0, 0), (0, DH - DO)))
    b1r = b1.reshape(1, DH)
    b2r = jnp.pad(b2, (0, DH - DO)).reshape(1, DH)

    h1 = pl.pallas_call(
        _tc_mm_body,
        out_shape=jax.ShapeDtypeStruct((R, DH), jnp.float32),
    )(xp, W1)

    degp = _sc_deg(dstp)

    dinvc = pl.pallas_call(
        _tc0_body,
        out_shape=jax.ShapeDtypeStruct((R // 128, 128), jnp.float32),
    )(degp).reshape(R, 1)

    hs1, dinv = pl.pallas_call(
        _tc1_body,
        out_shape=[
            jax.ShapeDtypeStruct((R, DH), jnp.float32),
            jax.ShapeDtypeStruct((R, DH), jnp.float32),
        ],
    )(h1, dinvc)

    msgp1 = _sc_msg(hs1, srcp, dstp)

    hs2 = pl.pallas_call(
        _tc2_body,
        out_shape=jax.ShapeDtypeStruct((R, DH), jnp.float32),
    )(msgp1, hs1, dinv, b1r, w2p)

    msgp2 = _sc_msg(hs2, srcp, dstp)

    out16 = pl.pallas_call(
        _tc3_body,
        out_shape=jax.ShapeDtypeStruct((R, DH), jnp.float32),
    )(msgp2, hs2, dinv, b2r)

    return out16[:N, :DO]


# speedup vs baseline: 57.8468x; 1.0020x over previous
import functools

import jax
import jax.numpy as jnp
from jax import lax
from jax.experimental import pallas as pl
from jax.experimental.pallas import tpu as pltpu
from jax.experimental.pallas import tpu_sc as plsc

N = 10000
R = 10240
E = 320000
NT = 32
NB = 80
EP = NT * NB * 128
DUMP = N
DH = 16
DO = 10

_MESH = plsc.VectorSubcoreMesh(
    core_axis_name="c", subcore_axis_name="s", num_cores=2, num_subcores=16
)


@functools.partial(
    pl.kernel,
    out_type=jax.ShapeDtypeStruct((NT, R // 128, 128), jnp.float32),
    mesh=_MESH,
    compiler_params=pltpu.CompilerParams(needs_layout_passes=False),
    scratch_types=[
        pltpu.VMEM((NB, 128), jnp.int32),
        pltpu.VMEM((R // 128, 128), jnp.float32),
    ],
)
def _sc_deg(dst_hbm, degp_hbm, dstv, ldeg):
    c = lax.axis_index("c")
    s = lax.axis_index("s")
    tid = c * 16 + s
    zeros = jnp.zeros((16,), jnp.float32)

    pltpu.sync_copy(dst_hbm.at[tid], dstv)

    def zero_row(i, carry):
        for kk in range(8):
            ldeg[i, pl.ds(kk * 16, 16)] = zeros
        return carry

    lax.fori_loop(0, R // 128, zero_row, 0)

    def accum(jj, carry):
        for kk in range(8):
            d = dstv[jj, pl.ds(kk * 16, 16)]
            cnt, last = plsc.scan_count(d)
            plsc.addupdate_scatter(
                ldeg,
                [lax.shift_right_logical(d, 7), lax.bitwise_and(d, 127)],
                cnt.astype(jnp.float32),
                mask=last,
            )
        return carry

    lax.fori_loop(0, NB, accum, 0)
    pltpu.sync_copy(ldeg, degp_hbm.at[tid])


@functools.partial(
    pl.kernel,
    out_type=jax.ShapeDtypeStruct((2, R, DH), jnp.float32),
    mesh=_MESH,
    compiler_params=pltpu.CompilerParams(use_tc_tiling_on_sc=False),
    scratch_types=[
        pltpu.VMEM((NB, 128), jnp.int32),
        pltpu.VMEM((NB, 128), jnp.int32),
        pltpu.VMEM((16, 128, DH), jnp.float32),
        pltpu.SemaphoreType.DMA,
        pltpu.SemaphoreType.DMA,
        pltpu.VMEM_SHARED((R, DH), jnp.float32),
        pltpu.VMEM_SHARED((R, DH), jnp.float32),
    ],
)
def _sc_msg(hs_hbm, src_hbm, dst_hbm, msgp_hbm, srcv, dstv, rows, gsem, ssem,
            acc, hs_sh):
    c = lax.axis_index("c")
    s = lax.axis_index("s")
    tid = c * 16 + s
    zeros = jnp.zeros((16,), jnp.float32)

    pltpu.sync_copy(src_hbm.at[tid], srcv)
    pltpu.sync_copy(dst_hbm.at[tid], dstv)
    base = s * (R // 16)
    pltpu.sync_copy(hs_hbm.at[pl.ds(base, R // 16)],
                    hs_sh.at[pl.ds(base, R // 16)])

    def zero_row(i, carry):
        rows[0, i, :] = zeros
        return carry

    lax.fori_loop(0, 128, zero_row, 0)

    def zero_sl(q, carry):
        pltpu.sync_copy(rows.at[0], acc.at[pl.ds(base + q * 128, 128)])
        return carry

    lax.fori_loop(0, R // 16 // 128, zero_sl, 0)
    plsc.subcore_barrier()

    def gather8(j, base, sem):
        for b in range(8):
            pltpu.async_copy(hs_sh.at[srcv.at[j + b]], rows.at[base + b], sem)

    def drain8(j, base, sem):
        for b in range(8):
            pltpu.make_async_copy(hs_sh.at[srcv.at[j + b]], rows.at[base + b],
                                  sem).wait()

    def scatter8(j, base, sem):
        for b in range(8):
            pltpu.async_copy(rows.at[base + b], acc.at[dstv.at[j + b]], sem,
                             add=True)

    def sdrain8(j, base, sem):
        for b in range(8):
            pltpu.make_async_copy(rows.at[base + b], acc.at[dstv.at[j + b]],
                                  sem).wait()

    gather8(0, 0, gsem)
    nsteps = NB // 16

    def step(o, carry):
        jA = o * 16
        jB = jA + 8
        drain8(jA, 0, gsem)
        scatter8(jA, 0, ssem)
        gather8(jB, 8, gsem)
        sdrain8(jA, 0, ssem)
        drain8(jB, 8, gsem)
        scatter8(jB, 8, ssem)

        @pl.when(o < nsteps - 1)
        def _():
            gather8(jA + 16, 0, gsem)

        sdrain8(jB, 8, ssem)
        return carry

    lax.fori_loop(0, nsteps, step, 0)

    plsc.subcore_barrier()

    @pl.when(s == 0)
    def _():
        pltpu.sync_copy(acc, msgp_hbm.at[c])


def _tc_mm_body(x_ref, w1_ref, h_ref):
    h_ref[...] = jnp.dot(x_ref[...], w1_ref[...],
                         preferred_element_type=jnp.float32)


def _tc0_body(degp_ref, dinv_ref):
    deg = jnp.sum(degp_ref[...], axis=0) + 1.0
    dinv_ref[...] = lax.rsqrt(jnp.maximum(deg, 1.0))


def _tc1_body(h_ref, dinvc_ref, hs1_ref, dinv_ref):
    dinv16 = jnp.broadcast_to(dinvc_ref[...], (R, DH))
    hs1_ref[...] = h_ref[...] * dinv16
    dinv_ref[...] = dinv16


def _tc2_body(msgp_ref, hs1_ref, dinv_ref, b1_ref, w2_ref, hs2_ref):
    msg = msgp_ref[0] + msgp_ref[1]
    pre = dinv_ref[...] * (msg + hs1_ref[...]) + b1_ref[...]
    out1 = jnp.maximum(pre, 0.0)
    h2 = jnp.dot(out1, w2_ref[...], preferred_element_type=jnp.float32)
    hs2_ref[...] = h2 * dinv_ref[...]


def _tc3_body(msgp_ref, hs2_ref, dinv_ref, b2_ref, out_ref):
    msg = msgp_ref[0] + msgp_ref[1]
    z = dinv_ref[...] * (msg + hs2_ref[...]) + b2_ref[...]
    col = lax.broadcasted_iota(jnp.int32, (1, DH), 1)
    z = jnp.where(col < DO, z, -1e30)
    m = jnp.max(z, axis=1, keepdims=True)
    ssum = jnp.sum(jnp.exp(z - m), axis=1, keepdims=True)
    out_ref[...] = z - m - jnp.log(ssum)


def kernel(x, edge_index, W1, b1, W2, b2):
    src = edge_index[0].astype(jnp.int32)
    dst = edge_index[1].astype(jnp.int32)
    pad = EP - E
    srcp = jnp.concatenate([src, jnp.zeros((pad,), jnp.int32)]).reshape(
        NT, NB, 128)
    dstp = jnp.concatenate([dst, jnp.full((pad,), DUMP, jnp.int32)]).reshape(
        NT, NB, 128)
    xp = jnp.pad(x.astype(jnp.float32), ((0, R - N), (0, 0)))
    w2p = jnp.pad(W2, ((0, 0), (0, DH - DO)))
    b1r = b1.reshape(1, DH)
    b2r = jnp.pad(b2, (0, DH - DO)).reshape(1, DH)

    h1 = pl.pallas_call(
        _tc_mm_body,
        out_shape=jax.ShapeDtypeStruct((R, DH), jnp.float32),
    )(xp, W1)

    degp = _sc_deg(dstp)

    dinvc = pl.pallas_call(
        _tc0_body,
        out_shape=jax.ShapeDtypeStruct((R // 128, 128), jnp.float32),
    )(degp).reshape(R, 1)

    hs1, dinv = pl.pallas_call(
        _tc1_body,
        out_shape=[
            jax.ShapeDtypeStruct((R, DH), jnp.float32),
            jax.ShapeDtypeStruct((R, DH), jnp.float32),
        ],
    )(h1, dinvc)

    msgp1 = _sc_msg(hs1, srcp, dstp)

    hs2 = pl.pallas_call(
        _tc2_body,
        out_shape=jax.ShapeDtypeStruct((R, DH), jnp.float32),
    )(msgp1, hs1, dinv, b1r, w2p)

    msgp2 = _sc_msg(hs2, srcp, dstp)

    out16 = pl.pallas_call(
        _tc3_body,
        out_shape=jax.ShapeDtypeStruct((R, DH), jnp.float32),
    )(msgp2, hs2, dinv, b2r)

    return out16[:N, :DO]
